# baseline (speedup 1.0000x reference)
import functools

import jax
import jax.numpy as jnp
from jax import lax
from jax.experimental import pallas as pl
from jax.experimental.pallas import tpu as pltpu
from jax.experimental.pallas import tpu_sc as plsc

VOCAB = 1000000
EMB = 64
BATCH = 16384
SEQ = 50
PH = 2 * SEQ
NV = EMB // 16

_info = plsc.get_sparse_core_info()
NC = _info.num_cores
NS = _info.num_subcores
NW = NC * NS
BPW = BATCH // NW
NBUF = 2
GRP = 1
NBLK = BPW // GRP
RPI = NBUF * GRP

_mesh = plsc.VectorSubcoreMesh(core_axis_name="c", subcore_axis_name="s")


_GATHER_DNUMS = lax.GatherDimensionNumbers(
    offset_dims=(), collapsed_slice_dims=(0,), start_index_map=(0,))


def _lane_perm(v, idx):
    return lax.gather(v, idx[:, None], _GATHER_DNUMS, slice_sizes=(1,),
                      mode=lax.GatherScatterMode.PROMISE_IN_BOUNDS)


def _lane_sum(v, lane):
    for sh in (8, 4, 2, 1):
        v = v + _lane_perm(v, lane ^ sh)
    return v


_SCHUNK = 12


def _pool_and_dot(buf, w, rbase):
    init = tuple(
        [jnp.maximum(buf[rbase, pl.ds(e * 16, 16)],
                     buf[rbase + SEQ - 1, pl.ds(e * 16, 16)])
         for e in range(NV)]
        + [jnp.maximum(buf[rbase + SEQ, pl.ds(e * 16, 16)],
                       buf[rbase + 2 * SEQ - 1, pl.ds(e * 16, 16)])
           for e in range(NV)])

    def sbody(g, carry):
        ps = list(carry)
        for k in range(_SCHUNK):
            s = rbase + 1 + g * _SCHUNK + k
            for e in range(NV):
                ps[e] = jnp.maximum(ps[e], buf[s, pl.ds(e * 16, 16)])
                ps[NV + e] = jnp.maximum(
                    ps[NV + e], buf[SEQ + s, pl.ds(e * 16, 16)])
        return tuple(ps)

    ps = lax.fori_loop(0, (SEQ - 2) // _SCHUNK, sbody, init)
    acc = ps[0] * w[0]
    for e in range(1, 2 * NV):
        acc = acc + ps[e] * w[e]
    return acc


@functools.partial(
    pl.kernel,
    mesh=_mesh,
    compiler_params=pltpu.CompilerParams(use_tc_tiling_on_sc=False),
    out_type=jax.ShapeDtypeStruct((BATCH,), jnp.float32),
    scratch_types=[
        pltpu.VMEM((NBLK, GRP * PH), jnp.int32),
        pltpu.VMEM((NBUF, GRP * PH, EMB), jnp.float32),
        pltpu.VMEM((BPW,), jnp.float32),
        pltpu.VMEM((9, 16), jnp.float32),
    ] + [pltpu.SemaphoreType.DMA] * NBUF,
)
def _pooled_lr(ph_hbm, table_hbm, wb_hbm, out_hbm,
               idx_v, bufs, lg, wv, *sems):
    wid = lax.axis_index("s") * NC + lax.axis_index("c")
    base = wid * NBLK

    pltpu.sync_copy(wb_hbm, wv)
    pltpu.sync_copy(ph_hbm.at[pl.ds(base, NBLK), :], idx_v)

    w = [wv[r, :] for r in range(2 * NV)]

    for j in range(NBUF - 1):
        pltpu.make_async_copy(
            table_hbm.at[idx_v.at[j]], bufs.at[j], sems[j]).start()

    lane = lax.broadcasted_iota(jnp.int32, (16,), 0)

    def body(g, vec):
        for b in range(NBUF):
            blk = g * NBUF + b
            nxt = blk + NBUF - 1
            tgt = (b + NBUF - 1) % NBUF

            @pl.when(nxt < NBLK)
            def _():
                pltpu.make_async_copy(
                    table_hbm.at[idx_v.at[nxt]], bufs.at[tgt],
                    sems[tgt]).start()

            pltpu.make_async_copy(
                table_hbm.at[idx_v.at[blk]], bufs.at[b], sems[b]).wait()
            for r in range(GRP):
                i = blk * GRP + r
                logit = _lane_sum(
                    _pool_and_dot(bufs.at[b], w, r * PH), lane)
                vec = jnp.where(lane == i % 16, logit, vec)

        @pl.when(g % (16 // RPI) == 16 // RPI - 1)
        def _():
            lg[pl.ds((g // (16 // RPI)) * 16, 16)] = vec

        return vec

    lax.fori_loop(0, NBLK // NBUF, body,
                  jnp.zeros((16,), dtype=jnp.float32))

    bvec = wv[2 * NV, :]
    one = jnp.full((16,), 1.0, dtype=jnp.float32)
    for j in range(BPW // 16):
        x = lg[pl.ds(j * 16, 16)]
        lg[pl.ds(j * 16, 16)] = one / (one + jnp.exp(-(x + bvec)))

    pltpu.sync_copy(lg, out_hbm.at[pl.ds(base, BPW)])


def kernel(premise, hypothesis, table, W, b):
    ph = jnp.concatenate(
        [premise.astype(jnp.int32), hypothesis.astype(jnp.int32)],
        axis=1).reshape(BATCH // GRP, GRP * PH)
    wb = jnp.concatenate(
        [W[:, 0], jnp.broadcast_to(b, (16,))]).reshape(9, 16).astype(jnp.float32)
    return _pooled_lr(ph, table, wb)


# speedup vs baseline: 1.0228x; 1.0228x over previous
import functools

import jax
import jax.numpy as jnp
from jax import lax
from jax.experimental import pallas as pl
from jax.experimental.pallas import tpu as pltpu
from jax.experimental.pallas import tpu_sc as plsc

VOCAB = 1000000
EMB = 64
BATCH = 16384
SEQ = 50
PH = 2 * SEQ
NV = EMB // 16

_info = plsc.get_sparse_core_info()
NC = _info.num_cores
NS = _info.num_subcores
NW = NC * NS
BPW = BATCH // NW
NBUF = 2
GRP = 1
NBLK = BPW // GRP
RPI = NBUF * GRP

_mesh = plsc.VectorSubcoreMesh(core_axis_name="c", subcore_axis_name="s")


_GATHER_DNUMS = lax.GatherDimensionNumbers(
    offset_dims=(), collapsed_slice_dims=(0,), start_index_map=(0,))


def _lane_perm(v, idx):
    return lax.gather(v, idx[:, None], _GATHER_DNUMS, slice_sizes=(1,),
                      mode=lax.GatherScatterMode.PROMISE_IN_BOUNDS)


def _lane_sum(v, lane):
    for sh in (8, 4, 2, 1):
        v = v + _lane_perm(v, lane ^ sh)
    return v


_SCHUNK = 8


def _pool_and_dot(buf, w, rbase):
    init = tuple(
        [jnp.maximum(buf[rbase, pl.ds(e * 16, 16)],
                     buf[rbase + SEQ - 1, pl.ds(e * 16, 16)])
         for e in range(NV)]
        + [jnp.maximum(buf[rbase + SEQ, pl.ds(e * 16, 16)],
                       buf[rbase + 2 * SEQ - 1, pl.ds(e * 16, 16)])
           for e in range(NV)])

    def sbody(g, carry):
        ps = list(carry)
        for k in range(_SCHUNK):
            s = rbase + 1 + g * _SCHUNK + k
            for e in range(NV):
                ps[e] = jnp.maximum(ps[e], buf[s, pl.ds(e * 16, 16)])
                ps[NV + e] = jnp.maximum(
                    ps[NV + e], buf[SEQ + s, pl.ds(e * 16, 16)])
        return tuple(ps)

    ps = lax.fori_loop(0, (SEQ - 2) // _SCHUNK, sbody, init)
    acc = ps[0] * w[0]
    for e in range(1, 2 * NV):
        acc = acc + ps[e] * w[e]
    return acc


@functools.partial(
    pl.kernel,
    mesh=_mesh,
    compiler_params=pltpu.CompilerParams(use_tc_tiling_on_sc=False),
    out_type=jax.ShapeDtypeStruct((BATCH,), jnp.float32),
    scratch_types=[
        pltpu.VMEM((NBLK, GRP * PH), jnp.int32),
        pltpu.VMEM((NBUF, GRP * PH, EMB), jnp.float32),
        pltpu.VMEM((BPW,), jnp.float32),
        pltpu.VMEM((9, 16), jnp.float32),
    ] + [pltpu.SemaphoreType.DMA] * NBUF,
)
def _pooled_lr(ph_hbm, table_hbm, wb_hbm, out_hbm,
               idx_v, bufs, lg, wv, *sems):
    wid = lax.axis_index("s") * NC + lax.axis_index("c")
    base = wid * NBLK

    pltpu.sync_copy(wb_hbm, wv)
    pltpu.sync_copy(ph_hbm.at[pl.ds(base, NBLK), :], idx_v)

    w = [wv[r, :] for r in range(2 * NV)]

    for j in range(NBUF - 1):
        pltpu.make_async_copy(
            table_hbm.at[idx_v.at[j]], bufs.at[j], sems[j]).start()

    lane = lax.broadcasted_iota(jnp.int32, (16,), 0)

    def body(g, vec):
        for b in range(NBUF):
            blk = g * NBUF + b
            nxt = blk + NBUF - 1
            tgt = (b + NBUF - 1) % NBUF

            @pl.when(nxt < NBLK)
            def _():
                pltpu.make_async_copy(
                    table_hbm.at[idx_v.at[nxt]], bufs.at[tgt],
                    sems[tgt]).start()

            pltpu.make_async_copy(
                table_hbm.at[idx_v.at[blk]], bufs.at[b], sems[b]).wait()
            for r in range(GRP):
                i = blk * GRP + r
                logit = _lane_sum(
                    _pool_and_dot(bufs.at[b], w, r * PH), lane)
                vec = jnp.where(lane == i % 16, logit, vec)

        @pl.when(g % (16 // RPI) == 16 // RPI - 1)
        def _():
            lg[pl.ds((g // (16 // RPI)) * 16, 16)] = vec

        return vec

    lax.fori_loop(0, NBLK // NBUF, body,
                  jnp.zeros((16,), dtype=jnp.float32))

    bvec = wv[2 * NV, :]
    one = jnp.full((16,), 1.0, dtype=jnp.float32)
    for j in range(BPW // 16):
        x = lg[pl.ds(j * 16, 16)]
        lg[pl.ds(j * 16, 16)] = one / (one + jnp.exp(-(x + bvec)))

    pltpu.sync_copy(lg, out_hbm.at[pl.ds(base, BPW)])


def kernel(premise, hypothesis, table, W, b):
    ph = jnp.concatenate(
        [premise.astype(jnp.int32), hypothesis.astype(jnp.int32)],
        axis=1).reshape(BATCH // GRP, GRP * PH)
    wb = jnp.concatenate(
        [W[:, 0], jnp.broadcast_to(b, (16,))]).reshape(9, 16).astype(jnp.float32)
    return _pooled_lr(ph, table, wb)


# speedup vs baseline: 1.1974x; 1.1708x over previous
import functools

import jax
import jax.numpy as jnp
from jax import lax
from jax.experimental import pallas as pl
from jax.experimental.pallas import tpu as pltpu
from jax.experimental.pallas import tpu_sc as plsc

VOCAB = 1000000
EMB = 64
BATCH = 16384
SEQ = 50
PH = 2 * SEQ
NV = EMB // 16

_info = plsc.get_sparse_core_info()
NC = _info.num_cores
NS = _info.num_subcores
NW = NC * NS
BPW = BATCH // NW
NBUF = 4
GRP = 1
NBLK = BPW // GRP
RPI = NBUF * GRP

_mesh = plsc.VectorSubcoreMesh(core_axis_name="c", subcore_axis_name="s")


_GATHER_DNUMS = lax.GatherDimensionNumbers(
    offset_dims=(), collapsed_slice_dims=(0,), start_index_map=(0,))


def _lane_perm(v, idx):
    return lax.gather(v, idx[:, None], _GATHER_DNUMS, slice_sizes=(1,),
                      mode=lax.GatherScatterMode.PROMISE_IN_BOUNDS)


def _lane_sum(v, lane):
    for sh in (8, 4, 2, 1):
        v = v + _lane_perm(v, lane ^ sh)
    return v


_SCHUNK = 8


def _pool_and_dot(buf, w, rbase):
    init = tuple(
        [jnp.maximum(buf[rbase, pl.ds(e * 16, 16)],
                     buf[rbase + SEQ - 1, pl.ds(e * 16, 16)])
         for e in range(NV)]
        + [jnp.maximum(buf[rbase + SEQ, pl.ds(e * 16, 16)],
                       buf[rbase + 2 * SEQ - 1, pl.ds(e * 16, 16)])
           for e in range(NV)])

    def sbody(g, carry):
        ps = list(carry)
        for k in range(_SCHUNK):
            s = rbase + 1 + g * _SCHUNK + k
            for e in range(NV):
                ps[e] = jnp.maximum(ps[e], buf[s, pl.ds(e * 16, 16)])
                ps[NV + e] = jnp.maximum(
                    ps[NV + e], buf[SEQ + s, pl.ds(e * 16, 16)])
        return tuple(ps)

    ps = lax.fori_loop(0, (SEQ - 2) // _SCHUNK, sbody, init)
    acc = ps[0] * w[0]
    for e in range(1, 2 * NV):
        acc = acc + ps[e] * w[e]
    return acc


@functools.partial(
    pl.kernel,
    mesh=_mesh,
    compiler_params=pltpu.CompilerParams(use_tc_tiling_on_sc=False),
    out_type=jax.ShapeDtypeStruct((BATCH,), jnp.float32),
    scratch_types=[
        pltpu.VMEM((NBLK, GRP * PH), jnp.int32),
        pltpu.VMEM((NBUF, GRP * PH, EMB), jnp.float32),
        pltpu.VMEM((BPW,), jnp.float32),
        pltpu.VMEM((9, 16), jnp.float32),
    ] + [pltpu.SemaphoreType.DMA] * NBUF,
)
def _pooled_lr(ph_hbm, table_hbm, wb_hbm, out_hbm,
               idx_v, bufs, lg, wv, *sems):
    wid = lax.axis_index("s") * NC + lax.axis_index("c")
    base = wid * NBLK

    pltpu.sync_copy(wb_hbm, wv)
    pltpu.sync_copy(ph_hbm.at[pl.ds(base, NBLK), :], idx_v)

    w = [wv[r, :] for r in range(2 * NV)]

    for j in range(NBUF - 1):
        pltpu.make_async_copy(
            table_hbm.at[idx_v.at[j]], bufs.at[j], sems[j]).start()

    lane = lax.broadcasted_iota(jnp.int32, (16,), 0)

    def body(g, vec):
        for b in range(NBUF):
            blk = g * NBUF + b
            nxt = blk + NBUF - 1
            tgt = (b + NBUF - 1) % NBUF

            @pl.when(nxt < NBLK)
            def _():
                pltpu.make_async_copy(
                    table_hbm.at[idx_v.at[nxt]], bufs.at[tgt],
                    sems[tgt]).start()

            pltpu.make_async_copy(
                table_hbm.at[idx_v.at[blk]], bufs.at[b], sems[b]).wait()
            for r in range(GRP):
                i = blk * GRP + r
                logit = _lane_sum(
                    _pool_and_dot(bufs.at[b], w, r * PH), lane)
                vec = jnp.where(lane == i % 16, logit, vec)

        @pl.when(g % (16 // RPI) == 16 // RPI - 1)
        def _():
            lg[pl.ds((g // (16 // RPI)) * 16, 16)] = vec

        return vec

    lax.fori_loop(0, NBLK // NBUF, body,
                  jnp.zeros((16,), dtype=jnp.float32))

    bvec = wv[2 * NV, :]
    one = jnp.full((16,), 1.0, dtype=jnp.float32)
    for j in range(BPW // 16):
        x = lg[pl.ds(j * 16, 16)]
        lg[pl.ds(j * 16, 16)] = one / (one + jnp.exp(-(x + bvec)))

    pltpu.sync_copy(lg, out_hbm.at[pl.ds(base, BPW)])


def kernel(premise, hypothesis, table, W, b):
    ph = jnp.concatenate(
        [premise.astype(jnp.int32), hypothesis.astype(jnp.int32)],
        axis=1).reshape(BATCH // GRP, GRP * PH)
    wb = jnp.concatenate(
        [W[:, 0], jnp.broadcast_to(b, (16,))]).reshape(9, 16).astype(jnp.float32)
    return _pooled_lr(ph, table, wb)


# speedup vs baseline: 1.2538x; 1.0471x over previous
import functools

import jax
import jax.numpy as jnp
from jax import lax
from jax.experimental import pallas as pl
from jax.experimental.pallas import tpu as pltpu
from jax.experimental.pallas import tpu_sc as plsc

VOCAB = 1000000
EMB = 64
BATCH = 16384
SEQ = 50
PH = 2 * SEQ
NV = EMB // 16

_info = plsc.get_sparse_core_info()
NC = _info.num_cores
NS = _info.num_subcores
NW = NC * NS
BPW = BATCH // NW
NBUF = 8
GRP = 1
NBLK = BPW // GRP
RPI = NBUF * GRP

_mesh = plsc.VectorSubcoreMesh(core_axis_name="c", subcore_axis_name="s")


_GATHER_DNUMS = lax.GatherDimensionNumbers(
    offset_dims=(), collapsed_slice_dims=(0,), start_index_map=(0,))


def _lane_perm(v, idx):
    return lax.gather(v, idx[:, None], _GATHER_DNUMS, slice_sizes=(1,),
                      mode=lax.GatherScatterMode.PROMISE_IN_BOUNDS)


def _lane_sum(v, lane):
    for sh in (8, 4, 2, 1):
        v = v + _lane_perm(v, lane ^ sh)
    return v


_SCHUNK = 8


def _pool_and_dot(buf, w, rbase):
    init = tuple(
        [jnp.maximum(buf[rbase, pl.ds(e * 16, 16)],
                     buf[rbase + SEQ - 1, pl.ds(e * 16, 16)])
         for e in range(NV)]
        + [jnp.maximum(buf[rbase + SEQ, pl.ds(e * 16, 16)],
                       buf[rbase + 2 * SEQ - 1, pl.ds(e * 16, 16)])
           for e in range(NV)])

    def sbody(g, carry):
        ps = list(carry)
        for k in range(_SCHUNK):
            s = rbase + 1 + g * _SCHUNK + k
            for e in range(NV):
                ps[e] = jnp.maximum(ps[e], buf[s, pl.ds(e * 16, 16)])
                ps[NV + e] = jnp.maximum(
                    ps[NV + e], buf[SEQ + s, pl.ds(e * 16, 16)])
        return tuple(ps)

    ps = lax.fori_loop(0, (SEQ - 2) // _SCHUNK, sbody, init)
    acc = ps[0] * w[0]
    for e in range(1, 2 * NV):
        acc = acc + ps[e] * w[e]
    return acc


@functools.partial(
    pl.kernel,
    mesh=_mesh,
    compiler_params=pltpu.CompilerParams(use_tc_tiling_on_sc=False),
    out_type=jax.ShapeDtypeStruct((BATCH,), jnp.float32),
    scratch_types=[
        pltpu.VMEM((NBLK, GRP * PH), jnp.int32),
        pltpu.VMEM((NBUF, GRP * PH, EMB), jnp.float32),
        pltpu.VMEM((BPW,), jnp.float32),
        pltpu.VMEM((9, 16), jnp.float32),
    ] + [pltpu.SemaphoreType.DMA] * NBUF,
)
def _pooled_lr(ph_hbm, table_hbm, wb_hbm, out_hbm,
               idx_v, bufs, lg, wv, *sems):
    wid = lax.axis_index("s") * NC + lax.axis_index("c")
    base = wid * NBLK

    pltpu.sync_copy(wb_hbm, wv)
    pltpu.sync_copy(ph_hbm.at[pl.ds(base, NBLK), :], idx_v)

    w = [wv[r, :] for r in range(2 * NV)]

    for j in range(NBUF - 1):
        pltpu.make_async_copy(
            table_hbm.at[idx_v.at[j]], bufs.at[j], sems[j]).start()

    lane = lax.broadcasted_iota(jnp.int32, (16,), 0)

    def body(g, vec):
        for b in range(NBUF):
            blk = g * NBUF + b
            nxt = blk + NBUF - 1
            tgt = (b + NBUF - 1) % NBUF

            @pl.when(nxt < NBLK)
            def _():
                pltpu.make_async_copy(
                    table_hbm.at[idx_v.at[nxt]], bufs.at[tgt],
                    sems[tgt]).start()

            pltpu.make_async_copy(
                table_hbm.at[idx_v.at[blk]], bufs.at[b], sems[b]).wait()
            for r in range(GRP):
                i = blk * GRP + r
                logit = _lane_sum(
                    _pool_and_dot(bufs.at[b], w, r * PH), lane)
                vec = jnp.where(lane == i % 16, logit, vec)

        @pl.when(g % (16 // RPI) == 16 // RPI - 1)
        def _():
            lg[pl.ds((g // (16 // RPI)) * 16, 16)] = vec

        return vec

    lax.fori_loop(0, NBLK // NBUF, body,
                  jnp.zeros((16,), dtype=jnp.float32))

    bvec = wv[2 * NV, :]
    one = jnp.full((16,), 1.0, dtype=jnp.float32)
    for j in range(BPW // 16):
        x = lg[pl.ds(j * 16, 16)]
        lg[pl.ds(j * 16, 16)] = one / (one + jnp.exp(-(x + bvec)))

    pltpu.sync_copy(lg, out_hbm.at[pl.ds(base, BPW)])


def kernel(premise, hypothesis, table, W, b):
    ph = jnp.concatenate(
        [premise.astype(jnp.int32), hypothesis.astype(jnp.int32)],
        axis=1).reshape(BATCH // GRP, GRP * PH)
    wb = jnp.concatenate(
        [W[:, 0], jnp.broadcast_to(b, (16,))]).reshape(9, 16).astype(jnp.float32)
    return _pooled_lr(ph, table, wb)
